# baseline (speedup 1.0000x reference)
import functools

import jax
import jax.numpy as jnp
from jax import lax
from jax.experimental import pallas as pl
from jax.experimental.pallas import tpu as pltpu
from jax.experimental.pallas import tpu_sc as plsc

NN = 10000
EE = 320000
DIN = 128
DH = 256
HALF = DH // 2
NC = 2
NS = 16
LL = 16
NP = 10240
RPT = NP // NS
K = 128
EPAD = 331776
MB = 3
CB = EPAD // (NS * MB * K)
DC = EPAD // (NC * NS * K)
RBLK = 2000

_f32 = jnp.float32


def _deg_body(dst_hbm, zeros_hbm, out_hbm, dst_v, ones_v, hist_sp):
    c = lax.axis_index("c")
    s = lax.axis_index("s")
    wid = c * NS + s
    pltpu.sync_copy(zeros_hbm, hist_sp.at[pl.ds(s * RPT, RPT)])
    pltpu.sync_copy(dst_hbm.at[wid], dst_v)
    for j in range(K // LL):
        ones_v[pl.ds(j * LL, LL)] = jnp.ones((LL,), _f32)
    plsc.subcore_barrier()

    def chunk(j, carry):
        pltpu.sync_copy(ones_v, hist_sp.at[dst_v.at[j]], add=True)
        return carry

    lax.fori_loop(0, DC, chunk, 0)
    plsc.subcore_barrier()
    pltpu.sync_copy(hist_sp.at[pl.ds(s * RPT, RPT)],
                    out_hbm.at[c].at[pl.ds(s * RPT, RPT)])


@jax.jit
def _deg_call(dst_deg, zeros1):
    mesh = plsc.VectorSubcoreMesh(core_axis_name="c", subcore_axis_name="s")
    return pl.kernel(
        _deg_body,
        out_type=jax.ShapeDtypeStruct((NC, NP), _f32),
        mesh=mesh,
        scratch_types=[
            pltpu.VMEM((DC, K), jnp.int32),
            pltpu.VMEM((K,), _f32),
            pltpu.VMEM_SHARED((NP,), _f32),
        ],
    )(dst_deg, zeros1)


def _msg_body(u_hbm, src_hbm, dst_hbm, zeros_hbm, out_hbm,
              src_v, dst_v, rows0, rows1, sg0, sg1, acc_sp):
    c = lax.axis_index("c")
    s = lax.axis_index("s")
    pltpu.sync_copy(zeros_hbm, acc_sp.at[pl.ds(s * RPT, RPT)])
    plsc.subcore_barrier()

    uc = u_hbm.at[c]
    rows = (rows0, rows1)
    sg = (sg0, sg1)

    def g_desc(j, b):
        return pltpu.make_async_copy(uc.at[src_v.at[j]], rows[b], sg[b])

    def blk(bi, carry):
        pltpu.sync_copy(src_hbm.at[s].at[bi], src_v)
        pltpu.sync_copy(dst_hbm.at[s].at[bi], dst_v)
        g_desc(0, 0).start()

        def pair(p, c2):
            for b in (0, 1):
                j = 2 * p + b
                ob = 1 - b
                g_desc(j, b).wait()

                @pl.when(j + 1 < CB)
                def _():
                    g_desc(j + 1, ob).start()

                pltpu.sync_copy(rows[b], acc_sp.at[dst_v.at[j]], add=True)

            return c2

        lax.fori_loop(0, CB // 2, pair, 0)
        return carry

    lax.fori_loop(0, MB, blk, 0)
    plsc.subcore_barrier()
    pltpu.sync_copy(acc_sp.at[pl.ds(s * RPT, RPT)],
                    out_hbm.at[c].at[pl.ds(s * RPT, RPT)])


@jax.jit
def _msg_call(u, src_msg, dst_msg, zeros2):
    mesh = plsc.VectorSubcoreMesh(core_axis_name="c", subcore_axis_name="s")
    return pl.kernel(
        _msg_body,
        out_type=jax.ShapeDtypeStruct((NC, NP, HALF), _f32),
        mesh=mesh,
        scratch_types=[
            pltpu.VMEM((CB, K), jnp.int32),
            pltpu.VMEM((CB, K), jnp.int32),
            pltpu.VMEM((K, HALF), _f32),
            pltpu.VMEM((K, HALF), _f32),
            pltpu.SemaphoreType.DMA,
            pltpu.SemaphoreType.DMA,
            pltpu.VMEM_SHARED((NP, HALF), _f32),
        ],
    )(u, src_msg, dst_msg, zeros2)


def _k1_body(x_ref, w_ref, p0_ref, p1_ref, u_ref, dinv_ref):
    deg = p0_ref[...] + p1_ref[...] + 1.0
    dinv = 1.0 / jnp.sqrt(deg)
    h = jnp.dot(x_ref[...], w_ref[...], preferred_element_type=_f32)
    u = h * dinv
    u_ref[0] = u[:, :HALF]
    u_ref[1] = u[:, HALF:]
    dinv_ref[...] = dinv


@jax.jit
def _k1_call(x, W1, p0, p1):
    grid = NN // RBLK
    return pl.pallas_call(
        _k1_body,
        grid=(grid,),
        in_specs=[
            pl.BlockSpec((RBLK, DIN), lambda i: (i, 0)),
            pl.BlockSpec((DIN, DH), lambda i: (0, 0)),
            pl.BlockSpec((RBLK, 1), lambda i: (i, 0)),
            pl.BlockSpec((RBLK, 1), lambda i: (i, 0)),
        ],
        out_specs=[
            pl.BlockSpec((NC, RBLK, HALF), lambda i: (0, i, 0)),
            pl.BlockSpec((RBLK, 1), lambda i: (i, 0)),
        ],
        out_shape=[
            jax.ShapeDtypeStruct((NC, NN, HALF), _f32),
            jax.ShapeDtypeStruct((NN, 1), _f32),
        ],
    )(x, W1, p0, p1)


def _k2_body(s_ref, u_ref, dinv_ref, b_ref, w_ref, u2_ref):
    sfull = jnp.concatenate([s_ref[0], s_ref[1]], axis=1)
    ufull = jnp.concatenate([u_ref[0], u_ref[1]], axis=1)
    dinv = dinv_ref[...]
    h1 = jnp.maximum(dinv * (sfull + ufull) + b_ref[...], 0.0)
    h2 = jnp.dot(h1, w_ref[...], preferred_element_type=_f32)
    u2 = h2 * dinv
    u2_ref[0] = u2[:, :HALF]
    u2_ref[1] = u2[:, HALF:]


@jax.jit
def _k2_call(s, u, dinv, b1, W2):
    grid = NN // RBLK
    return pl.pallas_call(
        _k2_body,
        grid=(grid,),
        in_specs=[
            pl.BlockSpec((NC, RBLK, HALF), lambda i: (0, i, 0)),
            pl.BlockSpec((NC, RBLK, HALF), lambda i: (0, i, 0)),
            pl.BlockSpec((RBLK, 1), lambda i: (i, 0)),
            pl.BlockSpec((1, DH), lambda i: (0, 0)),
            pl.BlockSpec((DH, DH), lambda i: (0, 0)),
        ],
        out_specs=pl.BlockSpec((NC, RBLK, HALF), lambda i: (0, i, 0)),
        out_shape=jax.ShapeDtypeStruct((NC, NN, HALF), _f32),
    )(s, u, dinv, b1, W2)


def _k3_body(t_ref, u2_ref, dinv_ref, b_ref, wo_ref, bo_ref, out_ref, acc_ref):
    i = pl.program_id(0)
    tfull = jnp.concatenate([t_ref[0], t_ref[1]], axis=1)
    ufull = jnp.concatenate([u2_ref[0], u2_ref[1]], axis=1)
    rows = jnp.maximum(dinv_ref[...] * (tfull + ufull) + b_ref[...], 0.0)
    part = jnp.sum(rows, axis=0, keepdims=True)

    @pl.when(i == 0)
    def _():
        acc_ref[...] = part

    @pl.when(i > 0)
    def _():
        acc_ref[...] = acc_ref[...] + part

    @pl.when(i == pl.num_programs(0) - 1)
    def _():
        g = acc_ref[...] / NN
        out_ref[...] = (jnp.dot(g, wo_ref[...], preferred_element_type=_f32,
                 precision=lax.Precision.HIGHEST)
                        + bo_ref[...])


@jax.jit
def _k3_call(t, u2, dinv, b2, Wo, bo):
    grid = NN // RBLK
    return pl.pallas_call(
        _k3_body,
        grid=(grid,),
        in_specs=[
            pl.BlockSpec((NC, RBLK, HALF), lambda i: (0, i, 0)),
            pl.BlockSpec((NC, RBLK, HALF), lambda i: (0, i, 0)),
            pl.BlockSpec((RBLK, 1), lambda i: (i, 0)),
            pl.BlockSpec((1, DH), lambda i: (0, 0)),
            pl.BlockSpec((DH, 1), lambda i: (0, 0)),
            pl.BlockSpec((1, 1), lambda i: (0, 0)),
        ],
        out_specs=pl.BlockSpec((1, 1), lambda i: (0, 0)),
        out_shape=jax.ShapeDtypeStruct((1, 1), _f32),
        scratch_shapes=[pltpu.VMEM((1, DH), _f32)],
    )(t, u2, dinv, b2, Wo, bo)


def kernel(x, edge_index, W1, b1, W2, b2, Wo, bo):
    src = edge_index[0]
    dst = edge_index[1]
    pad_e = EPAD - EE
    srcp = jnp.concatenate([src, jnp.zeros((pad_e,), jnp.int32)])
    dstp = jnp.concatenate([dst, jnp.full((pad_e,), NP - 1, jnp.int32)])
    src_msg = srcp.reshape(NS, MB, CB, K)
    dst_msg = dstp.reshape(NS, MB, CB, K)
    dst_deg = dstp.reshape(NC * NS, DC, K)
    zeros1 = jnp.zeros((RPT,), _f32)
    zeros2 = jnp.zeros((RPT, HALF), _f32)

    degp = _deg_call(dst_deg, zeros1)
    u, dinv = _k1_call(x, W1, degp[0, :NN, None], degp[1, :NN, None])
    s = _msg_call(u, src_msg, dst_msg, zeros2)
    u2 = _k2_call(s, u, dinv, b1[None, :], W2)
    t = _msg_call(u2, src_msg, dst_msg, zeros2)
    return _k3_call(t, u2, dinv, b2[None, :], Wo, bo[None, :])


# speedup vs baseline: 1.6224x; 1.6224x over previous
import functools

import jax
import jax.numpy as jnp
from jax import lax
from jax.experimental import pallas as pl
from jax.experimental.pallas import tpu as pltpu
from jax.experimental.pallas import tpu_sc as plsc

NN = 10000
EE = 320000
DIN = 128
DH = 256
HALF = DH // 2
NC = 2
NS = 16
LL = 16
NP = 10240
RPT = NP // NS
K = 128
EPAD = 323584
MB = 2
CB = EPAD // (NS * MB * K)
DC = EPAD // (NC * NS * K)
RBLK = 2000

_f32 = jnp.float32


def _deg_body(dst_hbm, zeros_hbm, out_hbm, dst_v, ones_v, hist_sp):
    c = lax.axis_index("c")
    s = lax.axis_index("s")
    wid = c * NS + s
    pltpu.sync_copy(zeros_hbm, hist_sp.at[pl.ds(s * RPT, RPT)])
    pltpu.sync_copy(dst_hbm.at[wid], dst_v)
    for j in range(K // LL):
        ones_v[pl.ds(j * LL, LL)] = jnp.ones((LL,), _f32)
    plsc.subcore_barrier()

    def chunk(j, carry):
        pltpu.sync_copy(ones_v, hist_sp.at[dst_v.at[j]], add=True)
        return carry

    lax.fori_loop(0, DC, chunk, 0)
    plsc.subcore_barrier()
    pltpu.sync_copy(hist_sp.at[pl.ds(s * RPT, RPT)],
                    out_hbm.at[c].at[pl.ds(s * RPT, RPT)])


@jax.jit
def _deg_call(dst_deg, zeros1):
    mesh = plsc.VectorSubcoreMesh(core_axis_name="c", subcore_axis_name="s")
    return pl.kernel(
        _deg_body,
        out_type=jax.ShapeDtypeStruct((NC, NP), _f32),
        mesh=mesh,
        scratch_types=[
            pltpu.VMEM((DC, K), jnp.int32),
            pltpu.VMEM((K,), _f32),
            pltpu.VMEM_SHARED((NP,), _f32),
        ],
    )(dst_deg, zeros1)


def _msg_body(u_hbm, src_hbm, dst_hbm, zeros_hbm, out_hbm,
              src_v, dst_v, rows0, sg0, acc_sp):
    c = lax.axis_index("c")
    s = lax.axis_index("s")
    pltpu.sync_copy(zeros_hbm, acc_sp.at[pl.ds(s * RPT, RPT)])
    plsc.subcore_barrier()

    uc = u_hbm.at[c]

    def blk(bi, carry):
        pltpu.sync_copy(src_hbm.at[s].at[bi], src_v)
        pltpu.sync_copy(dst_hbm.at[s].at[bi], dst_v)

        def chunk(j, c2):
            pltpu.async_copy(uc.at[src_v.at[j]], rows0, sg0).wait()
            pltpu.sync_copy(rows0, acc_sp.at[dst_v.at[j]], add=True)
            return c2

        lax.fori_loop(0, CB, chunk, 0)
        return carry

    lax.fori_loop(0, MB, blk, 0)
    plsc.subcore_barrier()
    pltpu.sync_copy(acc_sp.at[pl.ds(s * RPT, RPT)],
                    out_hbm.at[c].at[pl.ds(s * RPT, RPT)])


@jax.jit
def _msg_call(u, src_msg, dst_msg, zeros2):
    mesh = plsc.VectorSubcoreMesh(core_axis_name="c", subcore_axis_name="s")
    return pl.kernel(
        _msg_body,
        out_type=jax.ShapeDtypeStruct((NC, NP, HALF), _f32),
        mesh=mesh,
        scratch_types=[
            pltpu.VMEM((CB, K), jnp.int32),
            pltpu.VMEM((CB, K), jnp.int32),
            pltpu.VMEM((K, HALF), _f32),
            pltpu.SemaphoreType.DMA,
            pltpu.VMEM_SHARED((NP, HALF), _f32),
        ],
    )(u, src_msg, dst_msg, zeros2)


def _k1_body(x_ref, w_ref, p0_ref, p1_ref, u_ref, dinv_ref):
    deg = p0_ref[...] + p1_ref[...] + 1.0
    dinv = 1.0 / jnp.sqrt(deg)
    h = jnp.dot(x_ref[...], w_ref[...], preferred_element_type=_f32)
    u = h * dinv
    u_ref[0] = u[:, :HALF]
    u_ref[1] = u[:, HALF:]
    dinv_ref[...] = dinv


@jax.jit
def _k1_call(x, W1, p0, p1):
    grid = NN // RBLK
    return pl.pallas_call(
        _k1_body,
        grid=(grid,),
        in_specs=[
            pl.BlockSpec((RBLK, DIN), lambda i: (i, 0)),
            pl.BlockSpec((DIN, DH), lambda i: (0, 0)),
            pl.BlockSpec((RBLK, 1), lambda i: (i, 0)),
            pl.BlockSpec((RBLK, 1), lambda i: (i, 0)),
        ],
        out_specs=[
            pl.BlockSpec((NC, RBLK, HALF), lambda i: (0, i, 0)),
            pl.BlockSpec((RBLK, 1), lambda i: (i, 0)),
        ],
        out_shape=[
            jax.ShapeDtypeStruct((NC, NN, HALF), _f32),
            jax.ShapeDtypeStruct((NN, 1), _f32),
        ],
    )(x, W1, p0, p1)


def _k2_body(s_ref, u_ref, dinv_ref, b_ref, w_ref, u2_ref):
    sfull = jnp.concatenate([s_ref[0], s_ref[1]], axis=1)
    ufull = jnp.concatenate([u_ref[0], u_ref[1]], axis=1)
    dinv = dinv_ref[...]
    h1 = jnp.maximum(dinv * (sfull + ufull) + b_ref[...], 0.0)
    h2 = jnp.dot(h1, w_ref[...], preferred_element_type=_f32)
    u2 = h2 * dinv
    u2_ref[0] = u2[:, :HALF]
    u2_ref[1] = u2[:, HALF:]


@jax.jit
def _k2_call(s, u, dinv, b1, W2):
    grid = NN // RBLK
    return pl.pallas_call(
        _k2_body,
        grid=(grid,),
        in_specs=[
            pl.BlockSpec((NC, RBLK, HALF), lambda i: (0, i, 0)),
            pl.BlockSpec((NC, RBLK, HALF), lambda i: (0, i, 0)),
            pl.BlockSpec((RBLK, 1), lambda i: (i, 0)),
            pl.BlockSpec((1, DH), lambda i: (0, 0)),
            pl.BlockSpec((DH, DH), lambda i: (0, 0)),
        ],
        out_specs=pl.BlockSpec((NC, RBLK, HALF), lambda i: (0, i, 0)),
        out_shape=jax.ShapeDtypeStruct((NC, NN, HALF), _f32),
    )(s, u, dinv, b1, W2)


def _k3_body(t_ref, u2_ref, dinv_ref, b_ref, wo_ref, bo_ref, out_ref, acc_ref):
    i = pl.program_id(0)
    tfull = jnp.concatenate([t_ref[0], t_ref[1]], axis=1)
    ufull = jnp.concatenate([u2_ref[0], u2_ref[1]], axis=1)
    rows = jnp.maximum(dinv_ref[...] * (tfull + ufull) + b_ref[...], 0.0)
    part = jnp.sum(rows, axis=0, keepdims=True)

    @pl.when(i == 0)
    def _():
        acc_ref[...] = part

    @pl.when(i > 0)
    def _():
        acc_ref[...] = acc_ref[...] + part

    @pl.when(i == pl.num_programs(0) - 1)
    def _():
        g = acc_ref[...] / NN
        out_ref[...] = (jnp.dot(g, wo_ref[...], preferred_element_type=_f32,
                 precision=lax.Precision.HIGHEST)
                        + bo_ref[...])


@jax.jit
def _k3_call(t, u2, dinv, b2, Wo, bo):
    grid = NN // RBLK
    return pl.pallas_call(
        _k3_body,
        grid=(grid,),
        in_specs=[
            pl.BlockSpec((NC, RBLK, HALF), lambda i: (0, i, 0)),
            pl.BlockSpec((NC, RBLK, HALF), lambda i: (0, i, 0)),
            pl.BlockSpec((RBLK, 1), lambda i: (i, 0)),
            pl.BlockSpec((1, DH), lambda i: (0, 0)),
            pl.BlockSpec((DH, 1), lambda i: (0, 0)),
            pl.BlockSpec((1, 1), lambda i: (0, 0)),
        ],
        out_specs=pl.BlockSpec((1, 1), lambda i: (0, 0)),
        out_shape=jax.ShapeDtypeStruct((1, 1), _f32),
        scratch_shapes=[pltpu.VMEM((1, DH), _f32)],
    )(t, u2, dinv, b2, Wo, bo)


def kernel(x, edge_index, W1, b1, W2, b2, Wo, bo):
    src = edge_index[0]
    dst = edge_index[1]
    pad_e = EPAD - EE
    srcp = jnp.concatenate([src, jnp.zeros((pad_e,), jnp.int32)])
    dstp = jnp.concatenate([dst, jnp.full((pad_e,), NP - 1, jnp.int32)])
    src_msg = srcp.reshape(NS, MB, CB, K)
    dst_msg = dstp.reshape(NS, MB, CB, K)
    dst_deg = dstp.reshape(NC * NS, DC, K)
    zeros1 = jnp.zeros((RPT,), _f32)
    zeros2 = jnp.zeros((RPT, HALF), _f32)

    degp = _deg_call(dst_deg, zeros1)
    u, dinv = _k1_call(x, W1, degp[0, :NN, None], degp[1, :NN, None])
    s = _msg_call(u, src_msg, dst_msg, zeros2)
    u2 = _k2_call(s, u, dinv, b1[None, :], W2)
    t = _msg_call(u2, src_msg, dst_msg, zeros2)
    return _k3_call(t, u2, dinv, b2[None, :], Wo, bo[None, :])
